# baseline (speedup 1.0000x reference)
import functools

import numpy as np
import jax
import jax.numpy as jnp
from jax import lax
from jax.experimental import pallas as pl
from jax.experimental.pallas import tpu as pltpu
from jax.experimental.pallas import tpu_sc as plsc

N_NODES = 10000
N_EDGES = 320000
D_IN = 128
D_EDGE = 16
D_OUT = 128

NC = 2
NS = 16
NW = NC * NS
CHUNK = 64
N_CHUNKS = N_EDGES // CHUNK
NG_MAX = (N_CHUNKS + NW - 1) // NW
ZCH = 16
N_ZCH = N_NODES // ZCH


def _node_proj_body(x_ref, w_ref, b_ref, o_ref):
    o_ref[...] = jnp.dot(x_ref[...], w_ref[...],
                         preferred_element_type=jnp.float32) + b_ref[...]


def _edge_proj_body(e_ref, w_ref, o_ref):
    o_ref[...] = jnp.dot(e_ref[...], w_ref[...],
                         preferred_element_type=jnp.float32
                         ).astype(jnp.bfloat16)


def _apply_body(x_ref, p_ref, w1_ref, w2_ref, b_ref, o_ref):
    hn = p_ref[0] + p_ref[1]
    acc = jnp.dot(x_ref[...], w1_ref[...], preferred_element_type=jnp.float32)
    acc += jnp.dot(hn, w2_ref[...], preferred_element_type=jnp.float32)
    o_ref[...] = jnp.maximum(acc + b_ref[...], 0.0)


def _sc_segment_body(p_hbm, eb_hbm, src_hbm, dst_hbm, out_hbm,
                     srcv, dstv, prow, ebv, mv, zv,
                     isems, dsems, gsems, ssems, osem, acc):
    c = lax.axis_index("c")
    s = lax.axis_index("s")
    wid = c * NS + s

    for r in range(ZCH):
        for j in range(8):
            zv[pl.ds(r, 1), pl.ds(j * 16, 16)] = jnp.zeros((1, 16), jnp.float32)

    n_zero = (N_ZCH + NS - 1) // NS

    @pl.loop(0, n_zero)
    def _(g):
        cidx = s + NS * g

        @pl.when(cidx < N_ZCH)
        def _():
            pltpu.async_copy(zv, acc.at[pl.ds(cidx * ZCH, ZCH)], osem)

    @pl.loop(0, n_zero)
    def _(g):
        cidx = s + NS * g

        @pl.when(cidx < N_ZCH)
        def _():
            pltpu.make_async_copy(zv, acc.at[pl.ds(cidx * ZCH, ZCH)],
                                  osem).wait()

    plsc.subcore_barrier()

    def stage_a(g, i, checked=True):
        ch = wid + NW * g

        def body():
            pltpu.async_copy(src_hbm.at[pl.ds(ch, 1)], srcv[i], isems[i])
            pltpu.async_copy(dst_hbm.at[pl.ds(ch, 1)], dstv[i], isems[i])

        if checked:
            pl.when(ch < N_CHUNKS)(body)
        else:
            body()

    def stage_b(g, i, d, drain, checked=True):
        ch = wid + NW * g

        def body():
            pltpu.make_async_copy(src_hbm.at[pl.ds(ch, 1)], srcv[i],
                                  isems[i]).wait()
            pltpu.make_async_copy(dst_hbm.at[pl.ds(ch, 1)], dstv[i],
                                  isems[i]).wait()
            if drain:
                pltpu.make_async_copy(mv[d], acc.at[dstv[(i + 2) % 4].at[0]],
                                      ssems[d]).wait()
            pltpu.async_copy(eb_hbm.at[pl.ds(ch * CHUNK, CHUNK)], ebv[d],
                             dsems[d])
            pltpu.async_copy(p_hbm.at[srcv[i].at[0]], prow[d], gsems[d])

        if checked:
            pl.when(ch < N_CHUNKS)(body)
        else:
            body()

    def stage_c(g, i, d, checked=True):
        ch = wid + NW * g

        def body():
            pltpu.make_async_copy(eb_hbm.at[pl.ds(ch * CHUNK, CHUNK)], ebv[d],
                                  dsems[d]).wait()
            pltpu.make_async_copy(p_hbm.at[srcv[i].at[0]], prow[d],
                                  gsems[d]).wait()

            @pl.loop(0, CHUNK // 2)
            def _(t):
                r0 = pl.multiple_of(t * 2, 2)
                for j in range(8):
                    e2 = ebv[d][pl.ds(r0, 2), pl.ds(j * 16, 16)].astype(
                        jnp.float32)
                    sla = (pl.ds(r0, 1), pl.ds(j * 16, 16))
                    slb = (pl.ds(r0 + 1, 1), pl.ds(j * 16, 16))
                    pa = prow[d][sla]
                    pb = prow[d][slb]
                    mv[d][sla] = jnp.maximum(pa + e2[0:1, :], 0.0)
                    mv[d][slb] = jnp.maximum(pb + e2[1:2, :], 0.0)

            pltpu.async_copy(mv[d], acc.at[dstv[i].at[0]], ssems[d],
                             add=True)

        if checked:
            pl.when(ch < N_CHUNKS)(body)
        else:
            body()

    stage_a(0, 0, checked=False)
    stage_a(1, 1, checked=False)
    stage_b(0, 0, 0, drain=False, checked=False)
    stage_a(2, 2, checked=False)
    for b in range(4):
        g = b
        stage_b(g + 1, (b + 1) % 4, (b + 1) % 2, drain=(g >= 1),
                checked=False)
        stage_c(g, b % 4, b % 2, checked=False)
        stage_a(g + 3, (b + 3) % 4, checked=False)

    @pl.loop(4, ((NG_MAX + 3) // 4) * 4, step=4)
    def _(t):
        for b in range(4):
            g = t + b
            stage_b(g + 1, (b + 1) % 4, (b + 1) % 2, drain=True)
            stage_c(g, b % 4, b % 2)
            stage_a(g + 3, (b + 3) % 4)

    for g in range(NG_MAX - 3, NG_MAX):
        ch = wid + NW * g

        @pl.when(jnp.logical_and(ch < N_CHUNKS, ch + 2 * NW >= N_CHUNKS))
        def _():
            pltpu.make_async_copy(mv[g % 2], acc.at[dstv[g % 4].at[0]],
                                  ssems[g % 2]).wait()

    plsc.subcore_barrier()

    @pl.loop(0, n_zero)
    def _(g):
        cidx = s + NS * g

        @pl.when(cidx < N_ZCH)
        def _():
            pltpu.async_copy(acc.at[pl.ds(cidx * ZCH, ZCH)],
                             out_hbm.at[c, pl.ds(cidx * ZCH, ZCH)], osem)

    @pl.loop(0, n_zero)
    def _(g):
        cidx = s + NS * g

        @pl.when(cidx < N_ZCH)
        def _():
            pltpu.make_async_copy(acc.at[pl.ds(cidx * ZCH, ZCH)],
                                  out_hbm.at[c, pl.ds(cidx * ZCH, ZCH)],
                                  osem).wait()


@jax.jit
def kernel(nfeats, efeats, edge_index, W_msg, b_msg, W_apply, b_apply):
    edge_index = edge_index.astype(jnp.int32)
    src = edge_index[0].reshape(N_CHUNKS, CHUNK)
    dst = edge_index[1].reshape(N_CHUNKS, CHUNK)
    b_msg2 = b_msg.reshape(1, D_OUT)
    b_apply2 = b_apply.reshape(1, D_OUT)

    p = pl.pallas_call(
        _node_proj_body,
        out_shape=jax.ShapeDtypeStruct((N_NODES, D_OUT), jnp.float32),
    )(nfeats, W_msg[:D_IN], b_msg2)

    EBLK = 4000
    eb = pl.pallas_call(
        _edge_proj_body,
        grid=(N_EDGES // EBLK,),
        in_specs=[
            pl.BlockSpec((EBLK, D_EDGE), lambda i: (i, 0)),
            pl.BlockSpec((D_EDGE, D_OUT), lambda i: (0, 0)),
        ],
        out_specs=pl.BlockSpec((EBLK, D_OUT), lambda i: (i, 0)),
        out_shape=jax.ShapeDtypeStruct((N_EDGES, D_OUT), jnp.bfloat16),
    )(efeats, W_msg[D_IN:])

    mesh = plsc.VectorSubcoreMesh(core_axis_name="c", subcore_axis_name="s")
    sc_fn = pl.kernel(
        _sc_segment_body,
        out_type=jax.ShapeDtypeStruct((NC, N_NODES, D_OUT), jnp.float32),
        mesh=mesh,
        scratch_types=[
            [pltpu.VMEM((1, CHUNK), jnp.int32) for _ in range(4)],
            [pltpu.VMEM((1, CHUNK), jnp.int32) for _ in range(4)],
            [pltpu.VMEM((CHUNK, D_OUT), jnp.float32) for _ in range(2)],
            [pltpu.VMEM((CHUNK, D_OUT), jnp.bfloat16) for _ in range(2)],
            [pltpu.VMEM((CHUNK, D_OUT), jnp.float32) for _ in range(2)],
            pltpu.VMEM((ZCH, D_OUT), jnp.float32),
            [pltpu.SemaphoreType.DMA for _ in range(4)],
            [pltpu.SemaphoreType.DMA for _ in range(2)],
            [pltpu.SemaphoreType.DMA for _ in range(2)],
            [pltpu.SemaphoreType.DMA for _ in range(2)],
            pltpu.SemaphoreType.DMA,
            pltpu.VMEM_SHARED((N_NODES, D_OUT), jnp.float32),
        ],
    )
    partials = sc_fn(p, eb, src, dst)

    ABLK = 1000
    h = pl.pallas_call(
        _apply_body,
        grid=(N_NODES // ABLK,),
        in_specs=[
            pl.BlockSpec((ABLK, D_IN), lambda i: (i, 0)),
            pl.BlockSpec((NC, ABLK, D_OUT), lambda i: (0, i, 0)),
            pl.BlockSpec((D_IN, D_OUT), lambda i: (0, 0)),
            pl.BlockSpec((D_OUT, D_OUT), lambda i: (0, 0)),
            pl.BlockSpec((1, D_OUT), lambda i: (0, 0)),
        ],
        out_specs=pl.BlockSpec((ABLK, D_OUT), lambda i: (i, 0)),
        out_shape=jax.ShapeDtypeStruct((N_NODES, D_OUT), jnp.float32),
    )(nfeats, partials, W_apply[:D_IN], W_apply[D_IN:], b_apply2)
    return h


# speedup vs baseline: 1.2920x; 1.2920x over previous
import functools

import numpy as np
import jax
import jax.numpy as jnp
from jax import lax
from jax.experimental import pallas as pl
from jax.experimental.pallas import tpu as pltpu
from jax.experimental.pallas import tpu_sc as plsc

N_NODES = 10000
N_EDGES = 320000
D_IN = 128
D_EDGE = 16
D_OUT = 128

NC = 2
NS = 16
NW = NC * NS
CHUNK = 64
N_CHUNKS = N_EDGES // CHUNK
NG_MAX = (N_CHUNKS + NW - 1) // NW
NEB = 4
NIX = 8
ZCH = 16
N_ZCH = N_NODES // ZCH


def _node_proj_body(x_ref, w_ref, b_ref, o_ref):
    o_ref[...] = jnp.dot(x_ref[...], w_ref[...],
                         preferred_element_type=jnp.float32) + b_ref[...]


def _edge_proj_body(e_ref, w_ref, o_ref):
    o_ref[...] = jnp.dot(e_ref[...], w_ref[...],
                         preferred_element_type=jnp.float32)


def _apply_body(x_ref, p_ref, w1_ref, w2_ref, b_ref, o_ref):
    hn = p_ref[0] + p_ref[1]
    acc = jnp.dot(x_ref[...], w1_ref[...], preferred_element_type=jnp.float32)
    acc += jnp.dot(hn, w2_ref[...], preferred_element_type=jnp.float32)
    o_ref[...] = jnp.maximum(acc + b_ref[...], 0.0)


def _sc_segment_body(p_hbm, eb_hbm, src_hbm, dst_hbm, out_hbm,
                     srcv, dstv, ebv, zv,
                     isems, dsems, gsems, ssems, osem, acc):
    c = lax.axis_index("c")
    s = lax.axis_index("s")
    wid = c * NS + s

    for r in range(ZCH):
        for j in range(8):
            zv[pl.ds(r, 1), pl.ds(j * 16, 16)] = jnp.zeros((1, 16), jnp.float32)

    n_zero = (N_ZCH + NS - 1) // NS

    @pl.loop(0, n_zero)
    def _(g):
        cidx = s + NS * g

        @pl.when(cidx < N_ZCH)
        def _():
            pltpu.async_copy(zv, acc.at[pl.ds(cidx * ZCH, ZCH)], osem)

    @pl.loop(0, n_zero)
    def _(g):
        cidx = s + NS * g

        @pl.when(cidx < N_ZCH)
        def _():
            pltpu.make_async_copy(zv, acc.at[pl.ds(cidx * ZCH, ZCH)],
                                  osem).wait()

    plsc.subcore_barrier()

    def stage_a(g, i, checked=True):
        ch = wid + NW * g

        def body():
            pltpu.async_copy(src_hbm.at[pl.ds(ch, 1)], srcv[i], isems[i])
            pltpu.async_copy(dst_hbm.at[pl.ds(ch, 1)], dstv[i], isems[i])

        if checked:
            pl.when(ch < N_CHUNKS)(body)
        else:
            body()

    def stage_b(g, i, d, drain, checked=True):
        ch = wid + NW * g

        def body():
            if drain:
                pltpu.make_async_copy(ebv[d], acc.at[dstv[(i - NEB) % NIX]
                                                     .at[0]],
                                      ssems[d]).wait()
            pltpu.async_copy(eb_hbm.at[pl.ds(ch * CHUNK, CHUNK)], ebv[d],
                             dsems[d])

        if checked:
            pl.when(ch < N_CHUNKS)(body)
        else:
            body()

    def stage_b2(g, i, d, checked=True):
        ch = wid + NW * g

        def body():
            pltpu.make_async_copy(src_hbm.at[pl.ds(ch, 1)], srcv[i],
                                  isems[i]).wait()
            pltpu.make_async_copy(dst_hbm.at[pl.ds(ch, 1)], dstv[i],
                                  isems[i]).wait()
            pltpu.make_async_copy(eb_hbm.at[pl.ds(ch * CHUNK, CHUNK)], ebv[d],
                                  dsems[d]).wait()
            pltpu.async_copy(p_hbm.at[srcv[i].at[0]], ebv[d], gsems[d],
                             add=True)

        if checked:
            pl.when(ch < N_CHUNKS)(body)
        else:
            body()

    def stage_c(g, i, d, checked=True):
        ch = wid + NW * g

        def body():
            pltpu.make_async_copy(p_hbm.at[srcv[i].at[0]], ebv[d],
                                  gsems[d]).wait()

            @pl.loop(0, CHUNK)
            def _(r):
                for j in range(8):
                    sl = (pl.ds(r, 1), pl.ds(j * 16, 16))
                    ebv[d][sl] = jnp.maximum(ebv[d][sl], 0.0)

            pltpu.async_copy(ebv[d], acc.at[dstv[i].at[0]], ssems[d],
                             add=True)

        if checked:
            pl.when(ch < N_CHUNKS)(body)
        else:
            body()

    for k in range(4):
        stage_a(k, k, checked=False)
    stage_b(0, 0, 0, drain=False, checked=False)
    stage_b(1, 1, 1, drain=False, checked=False)
    stage_b2(0, 0, 0, checked=False)
    for b in range(8):
        g = b
        stage_b(g + 2, (b + 2) % NIX, (b + 2) % NEB, drain=(g >= 2),
                checked=(g + 2 >= 8))
        stage_b2(g + 1, (b + 1) % NIX, (b + 1) % NEB, checked=(g + 1 >= 8))
        stage_c(g, b % NIX, b % NEB, checked=False)
        stage_a(g + 4, (b + 4) % NIX, checked=(g + 4 >= 8))

    @pl.loop(8, ((NG_MAX + 7) // 8) * 8, step=8)
    def _(t):
        for b in range(8):
            g = t + b
            stage_b(g + 2, (b + 2) % NIX, (b + 2) % NEB, drain=True)
            stage_b2(g + 1, (b + 1) % NIX, (b + 1) % NEB)
            stage_c(g, b % NIX, b % NEB)
            stage_a(g + 4, (b + 4) % NIX)

    for g in range(NG_MAX - NEB - 1, NG_MAX):
        ch = wid + NW * g

        @pl.when(jnp.logical_and(ch < N_CHUNKS, ch + NEB * NW >= N_CHUNKS))
        def _():
            pltpu.make_async_copy(ebv[g % NEB], acc.at[dstv[g % NIX].at[0]],
                                  ssems[g % NEB]).wait()

    plsc.subcore_barrier()

    @pl.loop(0, n_zero)
    def _(g):
        cidx = s + NS * g

        @pl.when(cidx < N_ZCH)
        def _():
            pltpu.async_copy(acc.at[pl.ds(cidx * ZCH, ZCH)],
                             out_hbm.at[c, pl.ds(cidx * ZCH, ZCH)], osem)

    @pl.loop(0, n_zero)
    def _(g):
        cidx = s + NS * g

        @pl.when(cidx < N_ZCH)
        def _():
            pltpu.make_async_copy(acc.at[pl.ds(cidx * ZCH, ZCH)],
                                  out_hbm.at[c, pl.ds(cidx * ZCH, ZCH)],
                                  osem).wait()


@jax.jit
def kernel(nfeats, efeats, edge_index, W_msg, b_msg, W_apply, b_apply):
    edge_index = edge_index.astype(jnp.int32)
    src = edge_index[0].reshape(N_CHUNKS, CHUNK)
    dst = edge_index[1].reshape(N_CHUNKS, CHUNK)
    b_msg2 = b_msg.reshape(1, D_OUT)
    b_apply2 = b_apply.reshape(1, D_OUT)

    p = pl.pallas_call(
        _node_proj_body,
        out_shape=jax.ShapeDtypeStruct((N_NODES, D_OUT), jnp.float32),
    )(nfeats, W_msg[:D_IN], b_msg2)

    EBLK = 4000
    eb = pl.pallas_call(
        _edge_proj_body,
        grid=(N_EDGES // EBLK,),
        in_specs=[
            pl.BlockSpec((EBLK, D_EDGE), lambda i: (i, 0)),
            pl.BlockSpec((D_EDGE, D_OUT), lambda i: (0, 0)),
        ],
        out_specs=pl.BlockSpec((EBLK, D_OUT), lambda i: (i, 0)),
        out_shape=jax.ShapeDtypeStruct((N_EDGES, D_OUT), jnp.float32),
    )(efeats, W_msg[D_IN:])

    mesh = plsc.VectorSubcoreMesh(core_axis_name="c", subcore_axis_name="s")
    sc_fn = pl.kernel(
        _sc_segment_body,
        out_type=jax.ShapeDtypeStruct((NC, N_NODES, D_OUT), jnp.float32),
        mesh=mesh,
        scratch_types=[
            [pltpu.VMEM((1, CHUNK), jnp.int32) for _ in range(NIX)],
            [pltpu.VMEM((1, CHUNK), jnp.int32) for _ in range(NIX)],
            [pltpu.VMEM((CHUNK, D_OUT), jnp.float32)
             for _ in range(NEB)],
            pltpu.VMEM((ZCH, D_OUT), jnp.float32),
            [pltpu.SemaphoreType.DMA for _ in range(NIX)],
            [pltpu.SemaphoreType.DMA for _ in range(NEB)],
            [pltpu.SemaphoreType.DMA for _ in range(NEB)],
            [pltpu.SemaphoreType.DMA for _ in range(NEB)],
            pltpu.SemaphoreType.DMA,
            pltpu.VMEM_SHARED((N_NODES, D_OUT), jnp.float32),
        ],
    )
    partials = sc_fn(p, eb, src, dst)

    ABLK = 1000
    h = pl.pallas_call(
        _apply_body,
        grid=(N_NODES // ABLK,),
        in_specs=[
            pl.BlockSpec((ABLK, D_IN), lambda i: (i, 0)),
            pl.BlockSpec((NC, ABLK, D_OUT), lambda i: (0, i, 0)),
            pl.BlockSpec((D_IN, D_OUT), lambda i: (0, 0)),
            pl.BlockSpec((D_OUT, D_OUT), lambda i: (0, 0)),
            pl.BlockSpec((1, D_OUT), lambda i: (0, 0)),
        ],
        out_specs=pl.BlockSpec((ABLK, D_OUT), lambda i: (i, 0)),
        out_shape=jax.ShapeDtypeStruct((N_NODES, D_OUT), jnp.float32),
    )(nfeats, partials, W_apply[:D_IN], W_apply[D_IN:], b_apply2)
    return h


# speedup vs baseline: 1.3495x; 1.0445x over previous
import functools

import numpy as np
import jax
import jax.numpy as jnp
from jax import lax
from jax.experimental import pallas as pl
from jax.experimental.pallas import tpu as pltpu
from jax.experimental.pallas import tpu_sc as plsc

N_NODES = 10000
N_EDGES = 320000
D_IN = 128
D_EDGE = 16
D_OUT = 128

NC = 2
NS = 16
NW = NC * NS
CHUNK = 64
N_CHUNKS = N_EDGES // CHUNK
NG_MAX = (N_CHUNKS + NW - 1) // NW
NEB = 4
NIX = 8
ZCH = 16
N_ZCH = N_NODES // ZCH


def _proj_body(e_ref, x_ref, w_ref, b_ref, eb_ref, p_ref):
    w = w_ref[...]
    eb_ref[...] = jnp.dot(e_ref[...], w[D_IN:],
                          preferred_element_type=jnp.float32)
    p_ref[...] = jnp.dot(x_ref[...], w[:D_IN],
                         preferred_element_type=jnp.float32) + b_ref[...]


def _self_proj_body(x_ref, w_ref, b_ref, o_ref):
    o_ref[...] = jnp.dot(x_ref[...], w_ref[...][:D_IN],
                         preferred_element_type=jnp.float32) + b_ref[...]


def _apply_body(f1_ref, p_ref, w_ref, o_ref):
    hn = p_ref[0] + p_ref[1]
    acc = jnp.dot(hn, w_ref[...][D_IN:], preferred_element_type=jnp.float32)
    o_ref[...] = jnp.maximum(acc + f1_ref[...], 0.0)


def _sc_segment_body(p_hbm, eb_hbm, src_hbm, dst_hbm, out_hbm,
                     srcv, dstv, ebv, zv,
                     isems, dsems, gsems, ssems, osem, acc):
    c = lax.axis_index("c")
    s = lax.axis_index("s")
    wid = c * NS + s

    for r in range(ZCH):
        for j in range(8):
            zv[pl.ds(r, 1), pl.ds(j * 16, 16)] = jnp.zeros((1, 16), jnp.float32)

    n_zero = (N_ZCH + NS - 1) // NS

    @pl.loop(0, n_zero)
    def _(g):
        cidx = s + NS * g

        @pl.when(cidx < N_ZCH)
        def _():
            pltpu.async_copy(zv, acc.at[pl.ds(cidx * ZCH, ZCH)], osem)

    @pl.loop(0, n_zero)
    def _(g):
        cidx = s + NS * g

        @pl.when(cidx < N_ZCH)
        def _():
            pltpu.make_async_copy(zv, acc.at[pl.ds(cidx * ZCH, ZCH)],
                                  osem).wait()

    plsc.subcore_barrier()

    def stage_a(g, i, checked=True):
        ch = wid + NW * g

        def body():
            pltpu.async_copy(src_hbm.at[pl.ds(ch, 1)], srcv[i], isems[i])
            pltpu.async_copy(dst_hbm.at[pl.ds(ch, 1)], dstv[i], isems[i])

        if checked:
            pl.when(ch < N_CHUNKS)(body)
        else:
            body()

    def stage_b(g, i, d, drain, checked=True):
        ch = wid + NW * g

        def body():
            if drain:
                pltpu.make_async_copy(ebv[d], acc.at[dstv[(i - NEB) % NIX]
                                                     .at[0]],
                                      ssems[d]).wait()
            pltpu.async_copy(eb_hbm.at[pl.ds(ch * CHUNK, CHUNK)], ebv[d],
                             dsems[d])

        if checked:
            pl.when(ch < N_CHUNKS)(body)
        else:
            body()

    def stage_b2(g, i, d, checked=True):
        ch = wid + NW * g

        def body():
            pltpu.make_async_copy(src_hbm.at[pl.ds(ch, 1)], srcv[i],
                                  isems[i]).wait()
            pltpu.make_async_copy(dst_hbm.at[pl.ds(ch, 1)], dstv[i],
                                  isems[i]).wait()
            pltpu.make_async_copy(eb_hbm.at[pl.ds(ch * CHUNK, CHUNK)], ebv[d],
                                  dsems[d]).wait()
            pltpu.async_copy(p_hbm.at[srcv[i].at[0]], ebv[d], gsems[d],
                             add=True)

        if checked:
            pl.when(ch < N_CHUNKS)(body)
        else:
            body()

    def stage_c(g, i, d, checked=True):
        ch = wid + NW * g

        def body():
            pltpu.make_async_copy(p_hbm.at[srcv[i].at[0]], ebv[d],
                                  gsems[d]).wait()

            @pl.loop(0, CHUNK)
            def _(r):
                for j in range(8):
                    sl = (pl.ds(r, 1), pl.ds(j * 16, 16))
                    ebv[d][sl] = jnp.maximum(ebv[d][sl], 0.0)

            pltpu.async_copy(ebv[d], acc.at[dstv[i].at[0]], ssems[d],
                             add=True)

        if checked:
            pl.when(ch < N_CHUNKS)(body)
        else:
            body()

    for k in range(4):
        stage_a(k, k, checked=False)
    stage_b(0, 0, 0, drain=False, checked=False)
    stage_b(1, 1, 1, drain=False, checked=False)
    stage_b2(0, 0, 0, checked=False)
    for b in range(8):
        g = b
        stage_b(g + 2, (b + 2) % NIX, (b + 2) % NEB, drain=(g >= 2),
                checked=(g + 2 >= 8))
        stage_b2(g + 1, (b + 1) % NIX, (b + 1) % NEB, checked=(g + 1 >= 8))
        stage_c(g, b % NIX, b % NEB, checked=False)
        stage_a(g + 4, (b + 4) % NIX, checked=(g + 4 >= 8))

    @pl.loop(8, ((NG_MAX + 7) // 8) * 8, step=8)
    def _(t):
        for b in range(8):
            g = t + b
            stage_b(g + 2, (b + 2) % NIX, (b + 2) % NEB, drain=True)
            stage_b2(g + 1, (b + 1) % NIX, (b + 1) % NEB)
            stage_c(g, b % NIX, b % NEB)
            stage_a(g + 4, (b + 4) % NIX)

    for g in range(NG_MAX - NEB - 1, NG_MAX):
        ch = wid + NW * g

        @pl.when(jnp.logical_and(ch < N_CHUNKS, ch + NEB * NW >= N_CHUNKS))
        def _():
            pltpu.make_async_copy(ebv[g % NEB], acc.at[dstv[g % NIX].at[0]],
                                  ssems[g % NEB]).wait()

    plsc.subcore_barrier()

    @pl.loop(0, n_zero)
    def _(g):
        cidx = s + NS * g

        @pl.when(cidx < N_ZCH)
        def _():
            pltpu.async_copy(acc.at[pl.ds(cidx * ZCH, ZCH)],
                             out_hbm.at[c, pl.ds(cidx * ZCH, ZCH)], osem)

    @pl.loop(0, n_zero)
    def _(g):
        cidx = s + NS * g

        @pl.when(cidx < N_ZCH)
        def _():
            pltpu.make_async_copy(acc.at[pl.ds(cidx * ZCH, ZCH)],
                                  out_hbm.at[c, pl.ds(cidx * ZCH, ZCH)],
                                  osem).wait()


@jax.jit
def kernel(nfeats, efeats, edge_index, W_msg, b_msg, W_apply, b_apply):
    edge_index = edge_index.astype(jnp.int32)
    src = edge_index[0].reshape(N_CHUNKS, CHUNK)
    dst = edge_index[1].reshape(N_CHUNKS, CHUNK)
    b_msg2 = b_msg.reshape(1, D_OUT)
    b_apply2 = b_apply.reshape(1, D_OUT)

    GP = 50
    EBLK = N_EDGES // GP
    PBLK = N_NODES // GP
    eb, p = pl.pallas_call(
        _proj_body,
        grid=(GP,),
        in_specs=[
            pl.BlockSpec((EBLK, D_EDGE), lambda i: (i, 0)),
            pl.BlockSpec((PBLK, D_IN), lambda i: (i, 0)),
            pl.BlockSpec((D_IN + D_EDGE, D_OUT), lambda i: (0, 0)),
            pl.BlockSpec((1, D_OUT), lambda i: (0, 0)),
        ],
        out_specs=[
            pl.BlockSpec((EBLK, D_OUT), lambda i: (i, 0)),
            pl.BlockSpec((PBLK, D_OUT), lambda i: (i, 0)),
        ],
        out_shape=[
            jax.ShapeDtypeStruct((N_EDGES, D_OUT), jnp.float32),
            jax.ShapeDtypeStruct((N_NODES, D_OUT), jnp.float32),
        ],
    )(efeats, nfeats, W_msg, b_msg2)

    FBLK = 2000
    f1 = pl.pallas_call(
        _self_proj_body,
        grid=(N_NODES // FBLK,),
        in_specs=[
            pl.BlockSpec((FBLK, D_IN), lambda i: (i, 0)),
            pl.BlockSpec((D_IN + D_OUT, D_OUT), lambda i: (0, 0)),
            pl.BlockSpec((1, D_OUT), lambda i: (0, 0)),
        ],
        out_specs=pl.BlockSpec((FBLK, D_OUT), lambda i: (i, 0)),
        out_shape=jax.ShapeDtypeStruct((N_NODES, D_OUT), jnp.float32),
    )(nfeats, W_apply, b_apply2)

    mesh = plsc.VectorSubcoreMesh(core_axis_name="c", subcore_axis_name="s")
    sc_fn = pl.kernel(
        _sc_segment_body,
        out_type=jax.ShapeDtypeStruct((NC, N_NODES, D_OUT), jnp.float32),
        mesh=mesh,
        scratch_types=[
            [pltpu.VMEM((1, CHUNK), jnp.int32) for _ in range(NIX)],
            [pltpu.VMEM((1, CHUNK), jnp.int32) for _ in range(NIX)],
            [pltpu.VMEM((CHUNK, D_OUT), jnp.float32)
             for _ in range(NEB)],
            pltpu.VMEM((ZCH, D_OUT), jnp.float32),
            [pltpu.SemaphoreType.DMA for _ in range(NIX)],
            [pltpu.SemaphoreType.DMA for _ in range(NEB)],
            [pltpu.SemaphoreType.DMA for _ in range(NEB)],
            [pltpu.SemaphoreType.DMA for _ in range(NEB)],
            pltpu.SemaphoreType.DMA,
            pltpu.VMEM_SHARED((N_NODES, D_OUT), jnp.float32),
        ],
    )
    partials = sc_fn(p, eb, src, dst)

    ABLK = 1000
    h = pl.pallas_call(
        _apply_body,
        grid=(N_NODES // ABLK,),
        in_specs=[
            pl.BlockSpec((ABLK, D_OUT), lambda i: (i, 0)),
            pl.BlockSpec((NC, ABLK, D_OUT), lambda i: (0, i, 0)),
            pl.BlockSpec((D_IN + D_OUT, D_OUT), lambda i: (0, 0)),
        ],
        out_specs=pl.BlockSpec((ABLK, D_OUT), lambda i: (i, 0)),
        out_shape=jax.ShapeDtypeStruct((N_NODES, D_OUT), jnp.float32),
    )(f1, partials, W_apply)
    return h
